# baseline (speedup 1.0000x reference)
import math

import jax
import jax.numpy as jnp
from jax import lax
from jax.experimental import pallas as pl
from jax.experimental.pallas import tpu as pltpu
from jax.experimental.pallas import tpu_sc as plsc

T, D, E = 2048, 2048, 16
C = int(2 * math.ceil(T / (E // 4)))
TB = 256

NC, NS, L = 1, 16, 16
NW = NC * NS
TW = T // NW
NEG = float("-inf")


def _gates_body(x_ref, wg_ref, gates_ref):
    x = x_ref[...]
    wg = wg_ref[...]
    logits = jax.lax.dot_general(
        x, wg, (((1,), (1,)), ((), ())), preferred_element_type=jnp.float32
    )
    m = jnp.max(logits, axis=1, keepdims=True)
    ex = jnp.exp(logits - m)
    gates = ex / jnp.sum(ex, axis=1, keepdims=True)
    gates_ref[...] = gates


def _gates(x, wg, *, interpret=False):
    return pl.pallas_call(
        _gates_body,
        out_shape=jax.ShapeDtypeStruct((T, E), jnp.float32),
        interpret=interpret,
    )(x, wg)


def _take16(table, idx):
    dn = lax.GatherDimensionNumbers(
        offset_dims=(), collapsed_slice_dims=(0,), start_index_map=(0,)
    )
    return lax.gather(
        table, idx[:, None], dn, (1,), mode=lax.GatherScatterMode.PROMISE_IN_BOUNDS
    )


def _sc_phase1_body(
    gates_hbm,
    e1_hbm,
    e2_hbm,
    p1_hbm,
    p2_hbm,
    g1_hbm,
    g2_hbm,
    cnt1_hbm,
    cnt2_hbm,
    sumg_hbm,
    gates_v,
    e1_v,
    e2_v,
    p1_v,
    p2_v,
    g1_v,
    g2_v,
    cnt_v,
    sg_v,
):
    wid = lax.axis_index("s")
    base = wid * TW

    pltpu.sync_copy(gates_hbm.at[pl.ds(base * E, TW * E)], gates_v)

    lane = lax.iota(jnp.int32, L)
    zi = jnp.zeros((L,), jnp.int32)
    zf = jnp.zeros((L,), jnp.float32)
    cnt1 = zi
    cnt2 = zi
    sumg = zf
    acc_e1 = zi
    acc_e2 = zi
    acc_p1 = zi
    acc_p2 = zi
    acc_g1 = zf
    acc_g2 = zf
    for i in range(TW):
        g = gates_v[pl.ds(i * E, E)]
        sumg = sumg + g
        m1 = jnp.max(g)
        e1 = plsc.all_reduce_ffs(g == m1)
        m1v = lane == e1
        gx = jnp.where(m1v, jnp.float32(NEG), g)
        m2 = jnp.max(gx)
        e2 = plsc.all_reduce_ffs(gx == m2)
        m2v = lane == e2
        p1 = jnp.sum(jnp.where(m1v, cnt1, 0))
        p2 = jnp.sum(jnp.where(m2v, cnt2, 0))
        g1s = jnp.sum(jnp.where(m1v, g, 0.0))
        g2s = jnp.sum(jnp.where(m2v, g, 0.0))
        cnt1 = cnt1 + m1v.astype(jnp.int32)
        cnt2 = cnt2 + m2v.astype(jnp.int32)
        sel = lane == (i % L)
        acc_e1 = jnp.where(sel, e1, acc_e1)
        acc_e2 = jnp.where(sel, e2, acc_e2)
        acc_p1 = jnp.where(sel, p1, acc_p1)
        acc_p2 = jnp.where(sel, p2, acc_p2)
        acc_g1 = jnp.where(sel, g1s, acc_g1)
        acc_g2 = jnp.where(sel, g2s, acc_g2)
        if i % L == L - 1:
            grp = i // L
            sl = pl.ds(grp * L, L)
            e1_v[sl] = acc_e1
            e2_v[sl] = acc_e2
            p1_v[sl] = acc_p1
            p2_v[sl] = acc_p2
            g1_v[sl] = acc_g1
            g2_v[sl] = acc_g2

    pltpu.sync_copy(e1_v, e1_hbm.at[pl.ds(base, TW)])
    pltpu.sync_copy(e2_v, e2_hbm.at[pl.ds(base, TW)])
    pltpu.sync_copy(p1_v, p1_hbm.at[pl.ds(base, TW)])
    pltpu.sync_copy(p2_v, p2_hbm.at[pl.ds(base, TW)])
    pltpu.sync_copy(g1_v, g1_hbm.at[pl.ds(base, TW)])
    pltpu.sync_copy(g2_v, g2_hbm.at[pl.ds(base, TW)])
    cnt_v[...] = cnt1
    pltpu.sync_copy(cnt_v, cnt1_hbm.at[pl.ds(wid * E, E)])
    cnt_v[...] = cnt2
    pltpu.sync_copy(cnt_v, cnt2_hbm.at[pl.ds(wid * E, E)])
    sg_v[...] = sumg
    pltpu.sync_copy(sg_v, sumg_hbm.at[pl.ds(wid * E, E)])


def _sc_phase2_body(
    e1_hbm,
    e2_hbm,
    p1_hbm,
    p2_hbm,
    g1_hbm,
    g2_hbm,
    cnt1_hbm,
    cnt2_hbm,
    sumg_hbm,
    flat1_hbm,
    flat2_hbm,
    g1o_hbm,
    g2o_hbm,
    laux_hbm,
    e1_v,
    e2_v,
    p1_v,
    p2_v,
    g1_v,
    g2_v,
    f1_v,
    f2_v,
    g1o_v,
    g2o_v,
    c1_v,
    c2_v,
    sg_v,
    laux_v,
):
    wid = lax.axis_index("s")
    base = wid * TW

    pltpu.sync_copy(cnt1_hbm, c1_v)
    pltpu.sync_copy(cnt2_hbm, c2_v)
    b1 = jnp.zeros((L,), jnp.int32)
    b2 = jnp.zeros((L,), jnp.int32)
    t1 = jnp.zeros((L,), jnp.int32)
    for r in range(NW):
        row1 = c1_v[pl.ds(r * E, E)]
        row2 = c2_v[pl.ds(r * E, E)]
        before = jnp.int32(r) < wid
        b1 = b1 + jnp.where(before, row1, 0)
        b2 = b2 + jnp.where(before, row2, 0)
        t1 = t1 + row1

    @pl.when(wid == 0)
    def _():
        pltpu.sync_copy(sumg_hbm, sg_v)
        sg = jnp.zeros((L,), jnp.float32)
        for r in range(NW):
            sg = sg + sg_v[pl.ds(r * E, E)]
        la = jnp.sum(sg / T * t1.astype(jnp.float32) / T) * (E * E / (E // 4))
        laux_v[...] = jnp.zeros((L,), jnp.float32) + la
        pltpu.sync_copy(laux_v, laux_hbm)

    pltpu.sync_copy(e1_hbm.at[pl.ds(base, TW)], e1_v)
    pltpu.sync_copy(e2_hbm.at[pl.ds(base, TW)], e2_v)
    pltpu.sync_copy(p1_hbm.at[pl.ds(base, TW)], p1_v)
    pltpu.sync_copy(p2_hbm.at[pl.ds(base, TW)], p2_v)
    pltpu.sync_copy(g1_hbm.at[pl.ds(base, TW)], g1_v)
    pltpu.sync_copy(g2_hbm.at[pl.ds(base, TW)], g2_v)

    eps = jnp.float32(jnp.finfo(jnp.float32).eps)
    for grp in range(TW // L):
        sl = pl.ds(grp * L, L)
        e1g = e1_v[sl]
        e2g = e2_v[sl]
        loc1 = _take16(b1, e1g) + p1_v[sl]
        loc2 = _take16(b2, e2g) + p2_v[sl] + _take16(t1, e2g)
        k1 = loc1 < C
        k2 = loc2 < C
        g1k = jnp.where(k1, g1_v[sl], 0.0)
        g2k = jnp.where(k2, g2_v[sl], 0.0)
        den = jnp.maximum(g1k + g2k, eps)
        g1o_v[sl] = g1k / den
        g2o_v[sl] = g2k / den
        f1_v[sl] = jnp.where(k1, e1g * C + loc1, -1)
        f2_v[sl] = jnp.where(k2, e2g * C + loc2, -1)

    pltpu.sync_copy(f1_v, flat1_hbm.at[pl.ds(base, TW)])
    pltpu.sync_copy(f2_v, flat2_hbm.at[pl.ds(base, TW)])
    pltpu.sync_copy(g1o_v, g1o_hbm.at[pl.ds(base, TW)])
    pltpu.sync_copy(g2o_v, g2o_hbm.at[pl.ds(base, TW)])


def _sc_route(gates_flat):
    mesh = plsc.VectorSubcoreMesh(
        core_axis_name="c", subcore_axis_name="s", num_cores=NC, num_subcores=NS
    )
    ivec = lambda n: jax.ShapeDtypeStruct((n,), jnp.int32)
    fvec = lambda n: jax.ShapeDtypeStruct((n,), jnp.float32)
    ph1 = pl.kernel(
        _sc_phase1_body,
        out_type=[ivec(T), ivec(T), ivec(T), ivec(T), fvec(T), fvec(T),
                  ivec(NW * E), ivec(NW * E), fvec(NW * E)],
        mesh=mesh,
        scratch_types=[
            pltpu.VMEM((TW * E,), jnp.float32),
            pltpu.VMEM((TW,), jnp.int32),
            pltpu.VMEM((TW,), jnp.int32),
            pltpu.VMEM((TW,), jnp.int32),
            pltpu.VMEM((TW,), jnp.int32),
            pltpu.VMEM((TW,), jnp.float32),
            pltpu.VMEM((TW,), jnp.float32),
            pltpu.VMEM((L,), jnp.int32),
            pltpu.VMEM((L,), jnp.float32),
        ],
        compiler_params=pltpu.CompilerParams(needs_layout_passes=False),
    )
    e1a, e2a, p1a, p2a, g1a, g2a, c1, c2, sg = ph1(gates_flat)
    ph2 = pl.kernel(
        _sc_phase2_body,
        out_type=[ivec(T), ivec(T), fvec(T), fvec(T), fvec(L)],
        mesh=mesh,
        scratch_types=[
            pltpu.VMEM((TW,), jnp.int32),
            pltpu.VMEM((TW,), jnp.int32),
            pltpu.VMEM((TW,), jnp.int32),
            pltpu.VMEM((TW,), jnp.int32),
            pltpu.VMEM((TW,), jnp.float32),
            pltpu.VMEM((TW,), jnp.float32),
            pltpu.VMEM((TW,), jnp.int32),
            pltpu.VMEM((TW,), jnp.int32),
            pltpu.VMEM((TW,), jnp.float32),
            pltpu.VMEM((TW,), jnp.float32),
            pltpu.VMEM((NW * E,), jnp.int32),
            pltpu.VMEM((NW * E,), jnp.int32),
            pltpu.VMEM((NW * E,), jnp.float32),
            pltpu.VMEM((L,), jnp.float32),
        ],
        compiler_params=pltpu.CompilerParams(needs_layout_passes=False),
    )
    return ph2(e1a, e2a, p1a, p2a, g1a, g2a, c1, c2, sg)


def _write_body(flat1_ref, flat2_ref, g1_ref, g2_ref, cw_ref, mask_ref):
    f1 = flat1_ref[...]
    f2 = flat2_ref[...]
    g1 = g1_ref[...]
    g2 = g2_ref[...]
    k = jax.lax.broadcasted_iota(jnp.int32, (TB, E, C), 1) * C + (
        jax.lax.broadcasted_iota(jnp.int32, (TB, E, C), 2)
    )
    cw = jnp.where(k == f1, g1, 0.0) + jnp.where(k == f2, g2, 0.0)
    cw_ref[...] = cw
    mask_ref[...] = (cw != 0.0).astype(jnp.int8)


def _write(flat1, flat2, g1, g2, *, interpret=False):
    grid = (T // TB,)
    tok = pl.BlockSpec((TB, 1, 1), lambda i: (i, 0, 0))
    big = pl.BlockSpec((TB, E, C), lambda i: (i, 0, 0))
    return pl.pallas_call(
        _write_body,
        grid=grid,
        in_specs=[tok, tok, tok, tok],
        out_specs=[big, big],
        out_shape=[
            jax.ShapeDtypeStruct((T, E, C), jnp.float32),
            jax.ShapeDtypeStruct((T, E, C), jnp.int8),
        ],
        interpret=interpret,
    )(flat1, flat2, g1, g2)


def kernel(input, wg):
    gates = _gates(input, wg)
    flat1, flat2, g1, g2, laux16 = _sc_route(gates.reshape(T * E))
    cw, mask8 = _write(
        flat1.reshape(T, 1, 1),
        flat2.reshape(T, 1, 1),
        g1.reshape(T, 1, 1),
        g2.reshape(T, 1, 1),
    )
    return (laux16[0].reshape(()), cw, mask8.view(jnp.bool_))
